# baseline (speedup 1.0000x reference)
import jax
import jax.numpy as jnp
from jax import lax
from jax.experimental import pallas as pl
from jax.experimental.pallas import tpu as pltpu
from jax.experimental.pallas import tpu_sc as plsc

_N = 10000
_D = 128
_H = 16
_E = 320000

_NC = 2
_NS = 16
_NW = _NC * _NS
_B = 128
_NBUF = 8
_GBUF = 16
_NB = 80
_EPAD = _NW * _NB * _B
_NPAD = 10240
_TRASH = 10000
_RPW = _NPAD // _NS
_PK = _NPAD // 8

_mesh = plsc.VectorSubcoreMesh(core_axis_name="c", subcore_axis_name="s")


def _sc_counts_body(didx_hbm, sidx_hbm, cdst_hbm, csrc_hbm,
                    idx_d, idx_s, ones_v, zrow, cdst_sh, csrc_sh, sem_d, sem_s):
    c = lax.axis_index("c")
    s = lax.axis_index("s")
    wid = s * _NC + c

    def _fill(i, carry):
        zrow[pl.ds(2 * i, 2), :] = jnp.zeros((2, 16), jnp.bfloat16)
        return carry
    lax.fori_loop(0, _RPW // 2, _fill, 0)

    def _ofill(i, carry):
        ones_v[pl.ds(2 * i, 2), :] = jnp.ones((2, 16), jnp.bfloat16)
        return carry
    lax.fori_loop(0, _B // 2, _ofill, 0)

    pltpu.sync_copy(zrow, cdst_sh.at[pl.ds(s * _RPW, _RPW)])
    pltpu.sync_copy(zrow, csrc_sh.at[pl.ds(s * _RPW, _RPW)])
    pltpu.sync_copy(didx_hbm.at[wid], idx_d)
    pltpu.sync_copy(sidx_hbm.at[wid], idx_s)
    plsc.subcore_barrier()

    def _group(g, carry):
        base = g * _NBUF
        for j in range(_NBUF):
            pltpu.async_copy(ones_v, cdst_sh.at[idx_d.at[base + j]], sem_d, add=True)
            pltpu.async_copy(ones_v, csrc_sh.at[idx_s.at[base + j]], sem_s, add=True)
        for j in range(_NBUF):
            pltpu.make_async_copy(ones_v, cdst_sh.at[idx_d.at[base + j]], sem_d).wait()
            pltpu.make_async_copy(ones_v, csrc_sh.at[idx_s.at[base + j]], sem_s).wait()
        return carry
    lax.fori_loop(0, _NB // _NBUF, _group, 0)

    plsc.subcore_barrier()
    pltpu.sync_copy(cdst_sh.at[pl.ds(s * _RPW, _RPW)], cdst_hbm.at[c, pl.ds(s * _RPW, _RPW)])
    pltpu.sync_copy(csrc_sh.at[pl.ds(s * _RPW, _RPW)], csrc_hbm.at[c, pl.ds(s * _RPW, _RPW)])


_sc_counts = pl.kernel(
    _sc_counts_body,
    out_type=(jax.ShapeDtypeStruct((_NC, _NPAD, _H), jnp.bfloat16),
              jax.ShapeDtypeStruct((_NC, _NPAD, _H), jnp.bfloat16)),
    mesh=_mesh,
    compiler_params=pltpu.CompilerParams(use_tc_tiling_on_sc=False),
    scratch_types=[
        pltpu.VMEM((_NB, _B), jnp.int32),
        pltpu.VMEM((_NB, _B), jnp.int32),
        pltpu.VMEM((_B, _H), jnp.bfloat16),
        pltpu.VMEM((_RPW, _H), jnp.bfloat16),
        pltpu.VMEM_SHARED((_NPAD, _H), jnp.bfloat16),
        pltpu.VMEM_SHARED((_NPAD, _H), jnp.bfloat16),
        pltpu.SemaphoreType.DMA,
        pltpu.SemaphoreType.DMA,
    ],
)


def _sc_aggr_body(table_hbm, gidx_hbm, sidx_hbm, out_hbm,
                  idx_g, idx_s, rows, zrow, acc_sh, tbl_sh, gsem, ssem):
    c = lax.axis_index("c")
    s = lax.axis_index("s")
    wid = s * _NC + c

    def _zfill(i, carry):
        zrow[i, :] = jnp.zeros((16,), jnp.float32)
        return carry
    lax.fori_loop(0, _RPW, _zfill, 0)

    pltpu.sync_copy(zrow, acc_sh.at[pl.ds(s * _RPW, _RPW)])
    pltpu.sync_copy(table_hbm.at[pl.ds(s * _RPW, _RPW)], tbl_sh.at[pl.ds(s * _RPW, _RPW)])
    pltpu.sync_copy(gidx_hbm.at[wid], idx_g)
    pltpu.sync_copy(sidx_hbm.at[wid], idx_s)
    plsc.subcore_barrier()

    def _fire_g(g):
        for j in range(_GBUF):
            slot = (g % 2) * _GBUF + j
            pltpu.async_copy(tbl_sh.at[idx_g.at[g * _GBUF + j]], rows.at[slot], gsem)

    def _wait_g(g):
        for j in range(_GBUF):
            slot = (g % 2) * _GBUF + j
            pltpu.make_async_copy(tbl_sh.at[idx_g.at[g * _GBUF + j]], rows.at[slot], gsem).wait()

    def _fire_s(g):
        for j in range(_GBUF):
            slot = (g % 2) * _GBUF + j
            pltpu.async_copy(rows.at[slot], acc_sh.at[idx_s.at[g * _GBUF + j]], ssem, add=True)

    def _wait_s(g):
        for j in range(_GBUF):
            slot = (g % 2) * _GBUF + j
            pltpu.make_async_copy(rows.at[slot], acc_sh.at[idx_s.at[g * _GBUF + j]], ssem).wait()

    ng = _NB // _GBUF
    _fire_g(0)
    for g in range(ng):
        if g >= 1:
            _wait_s(g - 1)
        if g + 1 < ng:
            _fire_g(g + 1)
        _wait_g(g)
        _fire_s(g)
    _wait_s(ng - 1)

    plsc.subcore_barrier()
    pltpu.sync_copy(acc_sh.at[pl.ds(s * _RPW, _RPW)], out_hbm.at[c, pl.ds(s * _RPW, _RPW)])


_sc_aggr = pl.kernel(
    _sc_aggr_body,
    out_type=jax.ShapeDtypeStruct((_NC, _NPAD, _H), jnp.float32),
    mesh=_mesh,
    compiler_params=pltpu.CompilerParams(use_tc_tiling_on_sc=False),
    scratch_types=[
        pltpu.VMEM((_NB, _B), jnp.int32),
        pltpu.VMEM((_NB, _B), jnp.int32),
        pltpu.VMEM((2 * _GBUF, _B, _H), jnp.float32),
        pltpu.VMEM((_RPW, _H), jnp.float32),
        pltpu.VMEM_SHARED((_NPAD, _H), jnp.float32),
        pltpu.VMEM_SHARED((_NPAD, _H), jnp.float32),
        pltpu.SemaphoreType.DMA,
        pltpu.SemaphoreType.DMA,
    ],
)


def _tc_prep(edge_index, pad_tail):
    def body(e_ref, p_ref, s_ref, d_ref):
        s_ref[pl.ds(0, _E)] = e_ref[0, :]
        s_ref[pl.ds(_E, _EPAD - _E)] = p_ref[...]
        d_ref[pl.ds(0, _E)] = e_ref[1, :]
        d_ref[pl.ds(_E, _EPAD - _E)] = p_ref[...]

    return pl.pallas_call(
        body,
        out_shape=[jax.ShapeDtypeStruct((_EPAD,), jnp.int32)] * 2,
    )(edge_index, pad_tail)


_ROWBLK = 1280
_PBLK = _PK // 4


def _tc_proj(x_src_p, x_dst_p, Wm0, Ws0, b0, Ws1, b1):
    def body(xs_ref, xd_ref, wm0_ref, ws0_ref, b0_ref, ws1_ref, b1_ref, o_ref):
        xs = xs_ref[...]
        xd = xd_ref[...]
        p0 = jnp.dot(xs, wm0_ref[...], preferred_element_type=jnp.float32)
        sd = jnp.dot(xd, ws0_ref[...], preferred_element_type=jnp.float32) + b0_ref[...]
        ss = jnp.dot(xs, ws1_ref[...], preferred_element_type=jnp.float32) + b1_ref[...]
        pad = jnp.zeros((_ROWBLK, 128 - 3 * _H), jnp.float32)
        o_ref[...] = jnp.concatenate([p0, sd, ss, pad], axis=1)

    full = pl.BlockSpec((_D, _H), lambda i: (0, 0))
    bias = pl.BlockSpec((1, _H), lambda i: (0, 0))
    rows128 = pl.BlockSpec((_ROWBLK, _D), lambda i: (i, 0))
    return pl.pallas_call(
        body,
        grid=(_NPAD // _ROWBLK,),
        in_specs=[rows128, rows128, full, full, bias, full, bias],
        out_specs=rows128,
        out_shape=jax.ShapeDtypeStruct((_NPAD, _D), jnp.float32),
    )(x_src_p, x_dst_p, Wm0, Ws0, b0.reshape(1, _H), Ws1, b1.reshape(1, _H))


def _tc_layer(acc_p, cnt_p, s_term, Wblk):
    def body(a_ref, c_ref, s_ref, w_ref, h_ref, p_ref):
        a = a_ref[0] + a_ref[1]
        cnt = jnp.maximum(c_ref[0].astype(jnp.float32) + c_ref[1].astype(jnp.float32), 1.0)
        h = jnp.maximum(a / cnt + s_ref[...], 0.0)
        h_ref[...] = h
        p_ref[...] = jnp.dot(h, w_ref[...], preferred_element_type=jnp.float32)

    parts = pl.BlockSpec((_NC, _PBLK, 128), lambda i: (0, i, 0))
    packed = pl.BlockSpec((_PBLK, 128), lambda i: (i, 0))
    wspec = pl.BlockSpec((128, 128), lambda i: (0, 0))
    return pl.pallas_call(
        body,
        grid=(_PK // _PBLK,),
        in_specs=[parts, parts, packed, wspec],
        out_specs=[packed, packed],
        out_shape=[jax.ShapeDtypeStruct((_PK, 128), jnp.float32)] * 2,
    )(acc_p, cnt_p, s_term, Wblk)


def _tc_out(acc_p, cnt_p, h_dst, Wblk2, b2t):
    def body(a_ref, c_ref, h_ref, w_ref, b_ref, o_ref):
        a = a_ref[0] + a_ref[1]
        cnt = jnp.maximum(c_ref[0].astype(jnp.float32) + c_ref[1].astype(jnp.float32), 1.0)
        o_ref[...] = jnp.maximum(
            a / cnt + jnp.dot(h_ref[...], w_ref[...], preferred_element_type=jnp.float32)
            + b_ref[...], 0.0)

    parts = pl.BlockSpec((_NC, _PBLK, 128), lambda i: (0, i, 0))
    packed = pl.BlockSpec((_PBLK, 128), lambda i: (i, 0))
    wspec = pl.BlockSpec((128, 128), lambda i: (0, 0))
    bias = pl.BlockSpec((1, 128), lambda i: (0, 0))
    return pl.pallas_call(
        body,
        grid=(_PK // _PBLK,),
        in_specs=[parts, parts, packed, wspec, bias],
        out_specs=packed,
        out_shape=jax.ShapeDtypeStruct((_PK, 128), jnp.float32),
    )(acc_p, cnt_p, h_dst, Wblk2, b2t.reshape(1, 128))


def kernel(x_src, x_dst, edge_index, W_msg0, W_self0, b0,
           W_msg1, W_self1, b1, W_msg2, W_self2, b2):
    pad_tail = _TRASH + (jnp.arange(_EPAD - _E, dtype=jnp.int32) % (_NPAD - _N))
    eye8 = jnp.eye(8, dtype=jnp.float32)
    Wb1 = jnp.kron(eye8, W_msg1)
    Wb2 = jnp.kron(eye8, W_msg2)
    Wbs2 = jnp.kron(eye8, W_self2)
    b2t = jnp.tile(b2, 8)

    src_f, dst_f = _tc_prep(edge_index, pad_tail)
    src_w = src_f.reshape(_NW, _NB, _B)
    dst_w = dst_f.reshape(_NW, _NB, _B)

    cnt_dst, cnt_src = _sc_counts(dst_w, src_w)
    cnt_dst_p = cnt_dst.reshape(_NC, _PK, 128)
    cnt_src_p = cnt_src.reshape(_NC, _PK, 128)

    proj = _tc_proj(x_src, x_dst, W_msg0, W_self0, b0, W_self1, b1)
    p0 = proj[:, 0:_H]
    s_dst = proj[:, _H:2 * _H].reshape(_PK, 128)
    s_src = proj[:, 2 * _H:3 * _H].reshape(_PK, 128)

    acc0 = _sc_aggr(p0, src_w, dst_w)
    h_dst, p1 = _tc_layer(acc0.reshape(_NC, _PK, 128), cnt_dst_p, s_dst, Wb1)

    acc1 = _sc_aggr(p1.reshape(_NPAD, _H), dst_w, src_w)
    _, p2 = _tc_layer(acc1.reshape(_NC, _PK, 128), cnt_src_p, s_src, Wb2)

    acc2 = _sc_aggr(p2.reshape(_NPAD, _H), src_w, dst_w)
    out = _tc_out(acc2.reshape(_NC, _PK, 128), cnt_dst_p, h_dst, Wbs2, b2t)
    return out[:_N // 8].reshape(_N, _H)


# speedup vs baseline: 1.0750x; 1.0750x over previous
import jax
import jax.numpy as jnp
from jax import lax
from jax.experimental import pallas as pl
from jax.experimental.pallas import tpu as pltpu
from jax.experimental.pallas import tpu_sc as plsc

_N = 10000
_D = 128
_H = 16
_E = 320000

_NC = 2
_NS = 16
_NW = _NC * _NS
_B = 128
_NBUF = 8
_GBUF = 16
_NB = 80
_EPAD = _NW * _NB * _B
_NPAD = 10240
_TRASH = 10000
_RPW = _NPAD // _NS
_PK = _NPAD // 8

_mesh = plsc.VectorSubcoreMesh(core_axis_name="c", subcore_axis_name="s")


def _sc_counts_body(didx_hbm, sidx_hbm, cdst_hbm, csrc_hbm,
                    idx_d, idx_s, ones_v, zrow, cdst_sh, csrc_sh, sem_d, sem_s):
    c = lax.axis_index("c")
    s = lax.axis_index("s")
    wid = s * _NC + c

    def _fill(i, carry):
        zrow[pl.ds(2 * i, 2), :] = jnp.zeros((2, 16), jnp.bfloat16)
        return carry
    lax.fori_loop(0, _RPW // 2, _fill, 0)

    def _ofill(i, carry):
        ones_v[pl.ds(2 * i, 2), :] = jnp.ones((2, 16), jnp.bfloat16)
        return carry
    lax.fori_loop(0, _B // 2, _ofill, 0)

    pltpu.sync_copy(zrow, cdst_sh.at[pl.ds(s * _RPW, _RPW)])
    pltpu.sync_copy(zrow, csrc_sh.at[pl.ds(s * _RPW, _RPW)])
    pltpu.sync_copy(didx_hbm.at[wid], idx_d)
    pltpu.sync_copy(sidx_hbm.at[wid], idx_s)
    plsc.subcore_barrier()

    def _group(g, carry):
        base = g * _NBUF
        for j in range(_NBUF):
            pltpu.async_copy(ones_v, cdst_sh.at[idx_d.at[base + j]], sem_d, add=True)
            pltpu.async_copy(ones_v, csrc_sh.at[idx_s.at[base + j]], sem_s, add=True)
        for j in range(_NBUF):
            pltpu.make_async_copy(ones_v, cdst_sh.at[idx_d.at[base + j]], sem_d).wait()
            pltpu.make_async_copy(ones_v, csrc_sh.at[idx_s.at[base + j]], sem_s).wait()
        return carry
    lax.fori_loop(0, _NB // _NBUF, _group, 0)

    plsc.subcore_barrier()
    pltpu.sync_copy(cdst_sh.at[pl.ds(s * _RPW, _RPW)], cdst_hbm.at[c, pl.ds(s * _RPW, _RPW)])
    pltpu.sync_copy(csrc_sh.at[pl.ds(s * _RPW, _RPW)], csrc_hbm.at[c, pl.ds(s * _RPW, _RPW)])


_sc_counts = pl.kernel(
    _sc_counts_body,
    out_type=(jax.ShapeDtypeStruct((_NC, _NPAD, _H), jnp.bfloat16),
              jax.ShapeDtypeStruct((_NC, _NPAD, _H), jnp.bfloat16)),
    mesh=_mesh,
    compiler_params=pltpu.CompilerParams(use_tc_tiling_on_sc=False),
    scratch_types=[
        pltpu.VMEM((_NB, _B), jnp.int32),
        pltpu.VMEM((_NB, _B), jnp.int32),
        pltpu.VMEM((_B, _H), jnp.bfloat16),
        pltpu.VMEM((_RPW, _H), jnp.bfloat16),
        pltpu.VMEM_SHARED((_NPAD, _H), jnp.bfloat16),
        pltpu.VMEM_SHARED((_NPAD, _H), jnp.bfloat16),
        pltpu.SemaphoreType.DMA,
        pltpu.SemaphoreType.DMA,
    ],
)


def _sc_aggr_body(table_hbm, gidx_hbm, sidx_hbm, out_hbm,
                  idx_g, idx_s, rows, zrow, acc_sh, tbl_sh, gsem, ssem):
    c = lax.axis_index("c")
    s = lax.axis_index("s")
    wid = s * _NC + c

    def _zfill(i, carry):
        zrow[i, :] = jnp.zeros((16,), jnp.float32)
        return carry
    lax.fori_loop(0, _RPW, _zfill, 0)

    pltpu.sync_copy(zrow, acc_sh.at[pl.ds(s * _RPW, _RPW)])
    pltpu.sync_copy(table_hbm.at[pl.ds(s * _RPW, _RPW)], tbl_sh.at[pl.ds(s * _RPW, _RPW)])
    pltpu.sync_copy(gidx_hbm.at[wid], idx_g)
    pltpu.sync_copy(sidx_hbm.at[wid], idx_s)
    plsc.subcore_barrier()

    def _fire_g(g):
        for j in range(_GBUF):
            slot = (g % 2) * _GBUF + j
            pltpu.async_copy(tbl_sh.at[idx_g.at[g * _GBUF + j]], rows.at[slot], gsem)

    def _wait_g(g):
        for j in range(_GBUF):
            slot = (g % 2) * _GBUF + j
            pltpu.make_async_copy(tbl_sh.at[idx_g.at[g * _GBUF + j]], rows.at[slot], gsem).wait()

    def _fire_s(g):
        for j in range(_GBUF):
            slot = (g % 2) * _GBUF + j
            pltpu.async_copy(rows.at[slot], acc_sh.at[idx_s.at[g * _GBUF + j]], ssem, add=True)

    def _wait_s(g):
        for j in range(_GBUF):
            slot = (g % 2) * _GBUF + j
            pltpu.make_async_copy(rows.at[slot], acc_sh.at[idx_s.at[g * _GBUF + j]], ssem).wait()

    ng = _NB // _GBUF
    _fire_g(0)
    for g in range(ng):
        if g >= 1:
            _wait_s(g - 1)
        if g + 1 < ng:
            _fire_g(g + 1)
        _wait_g(g)
        _fire_s(g)
    _wait_s(ng - 1)

    plsc.subcore_barrier()
    pltpu.sync_copy(acc_sh.at[pl.ds(s * _RPW, _RPW)], out_hbm.at[c, pl.ds(s * _RPW, _RPW)])


_sc_aggr = pl.kernel(
    _sc_aggr_body,
    out_type=jax.ShapeDtypeStruct((_NC, _NPAD, _H), jnp.float32),
    mesh=_mesh,
    compiler_params=pltpu.CompilerParams(use_tc_tiling_on_sc=False),
    scratch_types=[
        pltpu.VMEM((_NB, _B), jnp.int32),
        pltpu.VMEM((_NB, _B), jnp.int32),
        pltpu.VMEM((2 * _GBUF, _B, _H), jnp.float32),
        pltpu.VMEM((_RPW, _H), jnp.float32),
        pltpu.VMEM_SHARED((_NPAD, _H), jnp.float32),
        pltpu.VMEM_SHARED((_NPAD, _H), jnp.float32),
        pltpu.SemaphoreType.DMA,
        pltpu.SemaphoreType.DMA,
    ],
)


def _tc_prep(edge_index, pad_tail):
    def body(e_ref, p_ref, s_ref, d_ref):
        s_ref[pl.ds(0, _E)] = e_ref[0, :]
        s_ref[pl.ds(_E, _EPAD - _E)] = p_ref[...]
        d_ref[pl.ds(0, _E)] = e_ref[1, :]
        d_ref[pl.ds(_E, _EPAD - _E)] = p_ref[...]

    return pl.pallas_call(
        body,
        out_shape=[jax.ShapeDtypeStruct((_EPAD,), jnp.int32)] * 2,
    )(edge_index, pad_tail)


_ROWBLK = 1280
_PBLK = _PK // 4


def _tc_proj(x_src_p, x_dst_p, Wm0, Ws0, b0, Ws1, b1):
    def body(xs_ref, xd_ref, wm0_ref, ws0_ref, b0_ref, ws1_ref, b1_ref,
             p0_ref, sd_ref, ss_ref):
        xs = xs_ref[...]
        xd = xd_ref[...]
        p0_ref[...] = jnp.dot(xs, wm0_ref[...], preferred_element_type=jnp.float32)
        sd_ref[...] = jnp.dot(xd, ws0_ref[...], preferred_element_type=jnp.float32) + b0_ref[...]
        ss_ref[...] = jnp.dot(xs, ws1_ref[...], preferred_element_type=jnp.float32) + b1_ref[...]

    full = pl.BlockSpec((_D, _H), lambda i: (0, 0))
    bias = pl.BlockSpec((1, _H), lambda i: (0, 0))
    rows128 = pl.BlockSpec((_ROWBLK, _D), lambda i: (i, 0))
    rows16 = pl.BlockSpec((_ROWBLK, _H), lambda i: (i, 0))
    return pl.pallas_call(
        body,
        grid=(_NPAD // _ROWBLK,),
        in_specs=[rows128, rows128, full, full, bias, full, bias],
        out_specs=[rows16, rows16, rows16],
        out_shape=[jax.ShapeDtypeStruct((_NPAD, _H), jnp.float32)] * 3,
    )(x_src_p, x_dst_p, Wm0, Ws0, b0.reshape(1, _H), Ws1, b1.reshape(1, _H))


def _tc_layer(acc_p, cnt_p, s_term, Wblk):
    def body(a_ref, c_ref, s_ref, w_ref, h_ref, p_ref):
        a = a_ref[0] + a_ref[1]
        cnt = jnp.maximum(c_ref[0].astype(jnp.float32) + c_ref[1].astype(jnp.float32), 1.0)
        h = jnp.maximum(a / cnt + s_ref[...], 0.0)
        h_ref[...] = h
        p_ref[...] = jnp.dot(h, w_ref[...], preferred_element_type=jnp.float32)

    parts = pl.BlockSpec((_NC, _PBLK, 128), lambda i: (0, i, 0))
    packed = pl.BlockSpec((_PBLK, 128), lambda i: (i, 0))
    wspec = pl.BlockSpec((128, 128), lambda i: (0, 0))
    return pl.pallas_call(
        body,
        grid=(_PK // _PBLK,),
        in_specs=[parts, parts, packed, wspec],
        out_specs=[packed, packed],
        out_shape=[jax.ShapeDtypeStruct((_PK, 128), jnp.float32)] * 2,
    )(acc_p, cnt_p, s_term, Wblk)


def _tc_out(acc_p, cnt_p, h_dst, Wblk2, b2t):
    def body(a_ref, c_ref, h_ref, w_ref, b_ref, o_ref):
        a = a_ref[0] + a_ref[1]
        cnt = jnp.maximum(c_ref[0].astype(jnp.float32) + c_ref[1].astype(jnp.float32), 1.0)
        o_ref[...] = jnp.maximum(
            a / cnt + jnp.dot(h_ref[...], w_ref[...], preferred_element_type=jnp.float32)
            + b_ref[...], 0.0)

    parts = pl.BlockSpec((_NC, _PBLK, 128), lambda i: (0, i, 0))
    packed = pl.BlockSpec((_PBLK, 128), lambda i: (i, 0))
    wspec = pl.BlockSpec((128, 128), lambda i: (0, 0))
    bias = pl.BlockSpec((1, 128), lambda i: (0, 0))
    return pl.pallas_call(
        body,
        grid=(_PK // _PBLK,),
        in_specs=[parts, parts, packed, wspec, bias],
        out_specs=packed,
        out_shape=jax.ShapeDtypeStruct((_PK, 128), jnp.float32),
    )(acc_p, cnt_p, h_dst, Wblk2, b2t.reshape(1, 128))


def kernel(x_src, x_dst, edge_index, W_msg0, W_self0, b0,
           W_msg1, W_self1, b1, W_msg2, W_self2, b2):
    pad_tail = _TRASH + (jnp.arange(_EPAD - _E, dtype=jnp.int32) % (_NPAD - _N))
    eye8 = jnp.eye(8, dtype=jnp.float32)
    Wb1 = jnp.kron(eye8, W_msg1)
    Wb2 = jnp.kron(eye8, W_msg2)
    Wbs2 = jnp.kron(eye8, W_self2)
    b2t = jnp.tile(b2, 8)

    src_f, dst_f = _tc_prep(edge_index, pad_tail)
    src_w = src_f.reshape(_NW, _NB, _B)
    dst_w = dst_f.reshape(_NW, _NB, _B)

    cnt_dst, cnt_src = _sc_counts(dst_w, src_w)
    cnt_dst_p = cnt_dst.reshape(_NC, _PK, 128)
    cnt_src_p = cnt_src.reshape(_NC, _PK, 128)

    p0, s_dst, s_src = _tc_proj(x_src, x_dst, W_msg0, W_self0, b0, W_self1, b1)
    s_dst = s_dst.reshape(_PK, 128)
    s_src = s_src.reshape(_PK, 128)

    acc0 = _sc_aggr(p0, src_w, dst_w)
    h_dst, p1 = _tc_layer(acc0.reshape(_NC, _PK, 128), cnt_dst_p, s_dst, Wb1)

    acc1 = _sc_aggr(p1.reshape(_NPAD, _H), dst_w, src_w)
    _, p2 = _tc_layer(acc1.reshape(_NC, _PK, 128), cnt_src_p, s_src, Wb2)

    acc2 = _sc_aggr(p2.reshape(_NPAD, _H), src_w, dst_w)
    out = _tc_out(acc2.reshape(_NC, _PK, 128), cnt_dst_p, h_dst, Wbs2, b2t)
    return out[:_N // 8].reshape(_N, _H)


# speedup vs baseline: 1.1805x; 1.0982x over previous
import jax
import jax.numpy as jnp
from jax import lax
from jax.experimental import pallas as pl
from jax.experimental.pallas import tpu as pltpu
from jax.experimental.pallas import tpu_sc as plsc

_N = 10000
_D = 128
_H = 16
_E = 320000

_NC = 2
_NS = 16
_NW = _NC * _NS
_B = 128
_NBUF = 8
_GBUF = 16
_NB = 80
_EPAD = _NW * _NB * _B
_NPAD = 10240
_TRASH = 10000
_RPW = _NPAD // _NS
_PK = _NPAD // 8

_mesh = plsc.VectorSubcoreMesh(core_axis_name="c", subcore_axis_name="s")


def _sc_counts_body(didx_hbm, sidx_hbm, cdst_hbm, csrc_hbm,
                    idx_d, idx_s, ones_v, zrow, cdst_sh, csrc_sh, sem_d, sem_s):
    c = lax.axis_index("c")
    s = lax.axis_index("s")
    wid = s * _NC + c

    pltpu.async_copy(didx_hbm.at[wid], idx_d, sem_d)
    pltpu.async_copy(sidx_hbm.at[wid], idx_s, sem_s)

    def _fill(i, carry):
        zrow[pl.ds(2 * i, 2), :] = jnp.zeros((2, 16), jnp.bfloat16)
        return carry
    lax.fori_loop(0, _RPW // 2, _fill, 0)

    def _ofill(i, carry):
        ones_v[pl.ds(2 * i, 2), :] = jnp.ones((2, 16), jnp.bfloat16)
        return carry
    lax.fori_loop(0, _B // 2, _ofill, 0)

    pltpu.sync_copy(zrow, cdst_sh.at[pl.ds(s * _RPW, _RPW)])
    pltpu.sync_copy(zrow, csrc_sh.at[pl.ds(s * _RPW, _RPW)])
    pltpu.make_async_copy(didx_hbm.at[wid], idx_d, sem_d).wait()
    pltpu.make_async_copy(sidx_hbm.at[wid], idx_s, sem_s).wait()
    plsc.subcore_barrier()

    def _group(g, carry):
        base = g * _NBUF
        for j in range(_NBUF):
            pltpu.async_copy(ones_v, cdst_sh.at[idx_d.at[base + j]], sem_d, add=True)
            pltpu.async_copy(ones_v, csrc_sh.at[idx_s.at[base + j]], sem_s, add=True)
        for j in range(_NBUF):
            pltpu.make_async_copy(ones_v, cdst_sh.at[idx_d.at[base + j]], sem_d).wait()
            pltpu.make_async_copy(ones_v, csrc_sh.at[idx_s.at[base + j]], sem_s).wait()
        return carry
    lax.fori_loop(0, _NB // _NBUF, _group, 0)

    plsc.subcore_barrier()
    pltpu.sync_copy(cdst_sh.at[pl.ds(s * _RPW, _RPW)], cdst_hbm.at[c, pl.ds(s * _RPW, _RPW)])
    pltpu.sync_copy(csrc_sh.at[pl.ds(s * _RPW, _RPW)], csrc_hbm.at[c, pl.ds(s * _RPW, _RPW)])


_sc_counts = pl.kernel(
    _sc_counts_body,
    out_type=(jax.ShapeDtypeStruct((_NC, _NPAD, _H), jnp.bfloat16),
              jax.ShapeDtypeStruct((_NC, _NPAD, _H), jnp.bfloat16)),
    mesh=_mesh,
    compiler_params=pltpu.CompilerParams(use_tc_tiling_on_sc=False),
    scratch_types=[
        pltpu.VMEM((_NB, _B), jnp.int32),
        pltpu.VMEM((_NB, _B), jnp.int32),
        pltpu.VMEM((_B, _H), jnp.bfloat16),
        pltpu.VMEM((_RPW, _H), jnp.bfloat16),
        pltpu.VMEM_SHARED((_NPAD, _H), jnp.bfloat16),
        pltpu.VMEM_SHARED((_NPAD, _H), jnp.bfloat16),
        pltpu.SemaphoreType.DMA,
        pltpu.SemaphoreType.DMA,
    ],
)


def _sc_aggr_body(table_hbm, gidx_hbm, sidx_hbm, out_hbm,
                  idx_g, idx_s, rows, zrow, acc_sh, tbl_sh, gsem, ssem):
    c = lax.axis_index("c")
    s = lax.axis_index("s")
    wid = s * _NC + c

    pltpu.async_copy(table_hbm.at[pl.ds(s * _RPW, _RPW)], tbl_sh.at[pl.ds(s * _RPW, _RPW)], gsem)
    pltpu.async_copy(gidx_hbm.at[wid], idx_g, ssem)
    pltpu.async_copy(sidx_hbm.at[wid], idx_s, ssem)

    def _zfill(i, carry):
        zrow[i, :] = jnp.zeros((16,), jnp.float32)
        return carry
    lax.fori_loop(0, _RPW, _zfill, 0)

    pltpu.sync_copy(zrow, acc_sh.at[pl.ds(s * _RPW, _RPW)])
    pltpu.make_async_copy(table_hbm.at[pl.ds(s * _RPW, _RPW)], tbl_sh.at[pl.ds(s * _RPW, _RPW)], gsem).wait()
    pltpu.make_async_copy(gidx_hbm.at[wid], idx_g, ssem).wait()
    pltpu.make_async_copy(sidx_hbm.at[wid], idx_s, ssem).wait()
    plsc.subcore_barrier()

    def _fire_g(g):
        for j in range(_GBUF):
            slot = (g % 2) * _GBUF + j
            pltpu.async_copy(tbl_sh.at[idx_g.at[g * _GBUF + j]], rows.at[slot], gsem)

    def _wait_g(g):
        for j in range(_GBUF):
            slot = (g % 2) * _GBUF + j
            pltpu.make_async_copy(tbl_sh.at[idx_g.at[g * _GBUF + j]], rows.at[slot], gsem).wait()

    def _fire_s(g):
        for j in range(_GBUF):
            slot = (g % 2) * _GBUF + j
            pltpu.async_copy(rows.at[slot], acc_sh.at[idx_s.at[g * _GBUF + j]], ssem, add=True)

    def _wait_s(g):
        for j in range(_GBUF):
            slot = (g % 2) * _GBUF + j
            pltpu.make_async_copy(rows.at[slot], acc_sh.at[idx_s.at[g * _GBUF + j]], ssem).wait()

    ng = _NB // _GBUF
    _fire_g(0)
    for g in range(ng):
        if g >= 1:
            _wait_s(g - 1)
        if g + 1 < ng:
            _fire_g(g + 1)
        _wait_g(g)
        _fire_s(g)
    _wait_s(ng - 1)

    plsc.subcore_barrier()
    pltpu.sync_copy(acc_sh.at[pl.ds(s * _RPW, _RPW)], out_hbm.at[c, pl.ds(s * _RPW, _RPW)])


_sc_aggr = pl.kernel(
    _sc_aggr_body,
    out_type=jax.ShapeDtypeStruct((_NC, _NPAD, _H), jnp.float32),
    mesh=_mesh,
    compiler_params=pltpu.CompilerParams(use_tc_tiling_on_sc=False),
    scratch_types=[
        pltpu.VMEM((_NB, _B), jnp.int32),
        pltpu.VMEM((_NB, _B), jnp.int32),
        pltpu.VMEM((2 * _GBUF, _B, _H), jnp.float32),
        pltpu.VMEM((_RPW, _H), jnp.float32),
        pltpu.VMEM_SHARED((_NPAD, _H), jnp.float32),
        pltpu.VMEM_SHARED((_NPAD, _H), jnp.float32),
        pltpu.SemaphoreType.DMA,
        pltpu.SemaphoreType.DMA,
    ],
)


def _tc_prep(edge_index, pad_tail):
    def body(e_ref, p_ref, s_ref, d_ref):
        s_ref[pl.ds(0, _E)] = e_ref[0, :]
        s_ref[pl.ds(_E, _EPAD - _E)] = p_ref[...]
        d_ref[pl.ds(0, _E)] = e_ref[1, :]
        d_ref[pl.ds(_E, _EPAD - _E)] = p_ref[...]

    return pl.pallas_call(
        body,
        out_shape=[jax.ShapeDtypeStruct((_EPAD,), jnp.int32)] * 2,
    )(edge_index, pad_tail)


_ROWBLK = 1280
_PBLK = _PK // 4


def _tc_proj(x_src_p, x_dst_p, Wm0, Ws0, b0, Ws1, b1):
    def body(xs_ref, xd_ref, wm0_ref, ws0_ref, b0_ref, ws1_ref, b1_ref,
             p0_ref, sd_ref, ss_ref):
        xs = xs_ref[...]
        xd = xd_ref[...]
        p0_ref[...] = jnp.dot(xs, wm0_ref[...], preferred_element_type=jnp.float32)
        sd_ref[...] = jnp.dot(xd, ws0_ref[...], preferred_element_type=jnp.float32) + b0_ref[...]
        ss_ref[...] = jnp.dot(xs, ws1_ref[...], preferred_element_type=jnp.float32) + b1_ref[...]

    full = pl.BlockSpec((_D, _H), lambda i: (0, 0))
    bias = pl.BlockSpec((1, _H), lambda i: (0, 0))
    rows128 = pl.BlockSpec((_ROWBLK, _D), lambda i: (i, 0))
    rows16 = pl.BlockSpec((_ROWBLK, _H), lambda i: (i, 0))
    return pl.pallas_call(
        body,
        grid=(_NPAD // _ROWBLK,),
        in_specs=[rows128, rows128, full, full, bias, full, bias],
        out_specs=[rows16, rows16, rows16],
        out_shape=[jax.ShapeDtypeStruct((_NPAD, _H), jnp.float32)] * 3,
    )(x_src_p, x_dst_p, Wm0, Ws0, b0.reshape(1, _H), Ws1, b1.reshape(1, _H))


def _tc_layer(acc_p, cnt_p, s_term, Wblk):
    def body(a_ref, c_ref, s_ref, w_ref, h_ref, p_ref):
        a = a_ref[0] + a_ref[1]
        cnt = jnp.maximum(c_ref[0].astype(jnp.float32) + c_ref[1].astype(jnp.float32), 1.0)
        h = jnp.maximum(a / cnt + s_ref[...], 0.0)
        h_ref[...] = h
        p_ref[...] = jnp.dot(h, w_ref[...], preferred_element_type=jnp.float32)

    parts = pl.BlockSpec((_NC, _PBLK, 128), lambda i: (0, i, 0))
    packed = pl.BlockSpec((_PBLK, 128), lambda i: (i, 0))
    wspec = pl.BlockSpec((128, 128), lambda i: (0, 0))
    return pl.pallas_call(
        body,
        grid=(_PK // _PBLK,),
        in_specs=[parts, parts, packed, wspec],
        out_specs=[packed, packed],
        out_shape=[jax.ShapeDtypeStruct((_PK, 128), jnp.float32)] * 2,
    )(acc_p, cnt_p, s_term, Wblk)


def _tc_out(acc_p, cnt_p, h_dst, Wblk2, b2t):
    def body(a_ref, c_ref, h_ref, w_ref, b_ref, o_ref):
        a = a_ref[0] + a_ref[1]
        cnt = jnp.maximum(c_ref[0].astype(jnp.float32) + c_ref[1].astype(jnp.float32), 1.0)
        o_ref[...] = jnp.maximum(
            a / cnt + jnp.dot(h_ref[...], w_ref[...], preferred_element_type=jnp.float32)
            + b_ref[...], 0.0)

    parts = pl.BlockSpec((_NC, _PBLK, 128), lambda i: (0, i, 0))
    packed = pl.BlockSpec((_PBLK, 128), lambda i: (i, 0))
    wspec = pl.BlockSpec((128, 128), lambda i: (0, 0))
    bias = pl.BlockSpec((1, 128), lambda i: (0, 0))
    return pl.pallas_call(
        body,
        grid=(_PK // _PBLK,),
        in_specs=[parts, parts, packed, wspec, bias],
        out_specs=packed,
        out_shape=jax.ShapeDtypeStruct((_PK, 128), jnp.float32),
    )(acc_p, cnt_p, h_dst, Wblk2, b2t.reshape(1, 128))


def kernel(x_src, x_dst, edge_index, W_msg0, W_self0, b0,
           W_msg1, W_self1, b1, W_msg2, W_self2, b2):
    pad_tail = _TRASH + (jnp.arange(_EPAD - _E, dtype=jnp.int32) % (_NPAD - _N))
    eye8 = jnp.eye(8, dtype=jnp.float32)
    Wb1 = jnp.kron(eye8, W_msg1)
    Wb2 = jnp.kron(eye8, W_msg2)
    Wbs2 = jnp.kron(eye8, W_self2)
    b2t = jnp.tile(b2, 8)

    src_f, dst_f = _tc_prep(edge_index, pad_tail)
    src_w = src_f.reshape(_NW, _NB, _B)
    dst_w = dst_f.reshape(_NW, _NB, _B)

    cnt_dst, cnt_src = _sc_counts(dst_w, src_w)
    cnt_dst_p = cnt_dst.reshape(_NC, _PK, 128)
    cnt_src_p = cnt_src.reshape(_NC, _PK, 128)

    p0, s_dst, s_src = _tc_proj(x_src, x_dst, W_msg0, W_self0, b0, W_self1, b1)
    s_dst = s_dst.reshape(_PK, 128)
    s_src = s_src.reshape(_PK, 128)

    acc0 = _sc_aggr(p0, src_w, dst_w)
    h_dst, p1 = _tc_layer(acc0.reshape(_NC, _PK, 128), cnt_dst_p, s_dst, Wb1)

    acc1 = _sc_aggr(p1.reshape(_NPAD, _H), dst_w, src_w)
    _, p2 = _tc_layer(acc1.reshape(_NC, _PK, 128), cnt_src_p, s_src, Wb2)

    acc2 = _sc_aggr(p2.reshape(_NPAD, _H), src_w, dst_w)
    out = _tc_out(acc2.reshape(_NC, _PK, 128), cnt_dst_p, h_dst, Wbs2, b2t)
    return out[:_N // 8].reshape(_N, _H)
